# baseline (speedup 1.0000x reference)
import jax
import jax.numpy as jnp
from jax.experimental import pallas as pl
from jax.experimental.pallas import tpu as pltpu

N = 4096
B = 512
K = 10


def _relu(x):
    return jnp.maximum(x, 0.0)


def _bdot(a, b):
    return jnp.dot(a.astype(jnp.bfloat16), b.astype(jnp.bfloat16),
                   preferred_element_type=jnp.float32)


def _chol_inv_t(gram):
    row = jax.lax.broadcasted_iota(jnp.int32, (K, K), 0)
    col = jax.lax.broadcasted_iota(jnp.int32, (K, K), 1)
    eye = (row == col).astype(jnp.float32)
    A = gram
    L = jnp.zeros((K, K), jnp.float32)
    for k in range(K):
        inv_s = jax.lax.rsqrt(A[k:k + 1, k:k + 1])
        lk = jnp.where(row[:, k:k + 1] >= k,
                       A[:, k:k + 1] * inv_s, 0.0)
        lk_t = jnp.where(col[k:k + 1, :] >= k,
                         A[k:k + 1, :] * inv_s, 0.0)
        L = L + jnp.where(col == k, lk, 0.0)
        A = A - lk * lk_t
    X = jnp.zeros((K, K), jnp.float32)
    for i in range(K):
        acc = jnp.dot(L, X, preferred_element_type=jnp.float32)
        xi = (eye[i:i + 1, :] - acc[i:i + 1, :]) / L[i:i + 1, i:i + 1]
        X = X + jnp.where(row == i, xi, 0.0)
    return X.T


def _mlp_gcn1_kernel(inputs_ref, adj_ref, w0, b0, w1, b1, w2, b2, w3, b3,
                     wg1, wg2, h_out, y_out, q_out, s_out, inv_out, ysum_out,
                     x1_scr, gram_scr, ysum_scr):
    i = pl.program_id(0)
    nsteps = pl.num_programs(0)

    @pl.when(i == 0)
    def _():
        x1_scr[...] = jnp.dot(inputs_ref[...], wg1[...],
                              preferred_element_type=jnp.float32
                              ).astype(jnp.bfloat16)

    x = inputs_ref[pl.ds(i * B, B), :].astype(jnp.bfloat16)
    t = jnp.dot(x, w0[...],
                preferred_element_type=jnp.float32).astype(jnp.bfloat16)
    t = _relu(t + b0[...])
    t = jnp.dot(t, w1[...],
                preferred_element_type=jnp.float32).astype(jnp.bfloat16)
    t = _relu(t + b1[...])
    t = jnp.dot(t, w2[...],
                preferred_element_type=jnp.float32).astype(jnp.bfloat16)
    t = _relu(t + b2[...])
    h = jnp.tanh(jnp.dot(t, w3[...], preferred_element_type=jnp.float32)
                 + b3[...])
    h_out[...] = h

    ab = adj_ref[...].astype(jnp.bfloat16)
    g = _relu(jnp.dot(ab, x1_scr[...], preferred_element_type=jnp.float32))
    y = _bdot(g, wg2[...])
    y_out[...] = y.astype(jnp.bfloat16)

    s = jnp.max(ab, axis=1, keepdims=True).astype(jnp.float32)
    s = jnp.maximum(s, 1e-30)
    c = (250.0 / s).astype(jnp.bfloat16)
    q_out[...] = (jnp.round(ab * c) - 125.0).astype(jnp.int8)
    s_out[...] = s

    gram_blk = jax.lax.dot_general(h, h, (((0,), (0,)), ((), ())),
                                   preferred_element_type=jnp.float32)
    ysum_blk = jnp.sum(y, axis=0, keepdims=True)

    @pl.when(i == 0)
    def _():
        gram_scr[...] = gram_blk
        ysum_scr[...] = ysum_blk

    @pl.when(i > 0)
    def _():
        gram_scr[...] += gram_blk
        ysum_scr[...] += ysum_blk

    @pl.when(i == nsteps - 1)
    def _():
        row = jax.lax.broadcasted_iota(jnp.int32, (K, K), 0)
        col = jax.lax.broadcasted_iota(jnp.int32, (K, K), 1)
        gram = gram_scr[...] + 1e-6 * (row == col).astype(jnp.float32)
        inv_out[...] = _chol_inv_t(gram)
        ysum_out[...] = ysum_scr[...]


def _ortho_gcn2_kernel(h_ref, y_ref, inv_ref, ysum_ref, s_ref, q_ref,
                       ortho_out, g_out):
    i = pl.program_id(0)
    hb = h_ref[pl.ds(i * B, B), :]
    ortho_out[...] = 64.0 * jnp.dot(hb, inv_ref[...],
                                    preferred_element_type=jnp.float32)
    dot = jnp.dot(q_ref[...].astype(jnp.bfloat16), y_ref[...],
                  preferred_element_type=jnp.float32)
    alpha = s_ref[...] * (1.0 / 250.0)
    g_out[...] = alpha * (dot + 125.0 * ysum_ref[...])


@jax.jit
def kernel(inputs, adj, Ws0, bs0, Ws1, bs1, Ws2, bs2, Ws3, bs3, Wg1, Wg2):
    f32 = jnp.float32
    bf16 = jnp.bfloat16
    w0b = Ws0.astype(bf16)
    w1b = Ws1.astype(bf16)
    w2p = jnp.pad(Ws2, ((0, 0), (0, 14))).astype(bf16)
    b2p = jnp.pad(bs2, (0, 14)).reshape(1, -1).astype(bf16)
    w3p = jnp.pad(Ws3, ((0, 14), (0, 0))).astype(bf16)
    b0 = bs0.reshape(1, -1).astype(bf16)
    b1 = bs1.reshape(1, -1).astype(bf16)
    b3 = bs3.reshape(1, -1)

    grid = N // B
    full = lambda s: pl.BlockSpec(s, lambda i: (0, 0))
    rows = lambda w: pl.BlockSpec((B, w), lambda i: (i, 0))

    h, y, q, s, inv_lt, ysum = pl.pallas_call(
        _mlp_gcn1_kernel,
        grid=(grid,),
        in_specs=[
            full((N, 128)),
            rows(N),
            full((128, 1024)), full((1, 1024)),
            full((1024, 512)), full((1, 512)),
            full((512, 64)), full((1, 64)),
            full((64, K)), full((1, K)),
            full((128, 64)),
            full((64, K)),
        ],
        out_specs=[rows(K), rows(K), rows(N), rows(1),
                   full((K, K)), full((1, K))],
        out_shape=[jax.ShapeDtypeStruct((N, K), f32),
                   jax.ShapeDtypeStruct((N, K), jnp.bfloat16),
                   jax.ShapeDtypeStruct((N, N), jnp.int8),
                   jax.ShapeDtypeStruct((N, 1), f32),
                   jax.ShapeDtypeStruct((K, K), f32),
                   jax.ShapeDtypeStruct((1, K), f32)],
        scratch_shapes=[pltpu.VMEM((N, 64), jnp.bfloat16),
                        pltpu.VMEM((K, K), f32),
                        pltpu.VMEM((1, K), f32)],
    )(inputs, adj, w0b, b0, w1b, b1, w2p, b2p, w3p, b3, Wg1, Wg2)

    ortho, out_g = pl.pallas_call(
        _ortho_gcn2_kernel,
        grid=(grid,),
        in_specs=[full((N, K)), full((N, K)), full((K, K)), full((1, K)),
                  rows(1), rows(N)],
        out_specs=[rows(K), rows(K)],
        out_shape=[jax.ShapeDtypeStruct((N, K), f32),
                   jax.ShapeDtypeStruct((N, K), f32)],
    )(h, y, inv_lt, ysum, s, q)

    return (ortho, out_g)


# speedup vs baseline: 1.2531x; 1.2531x over previous
import jax
import jax.numpy as jnp
from jax.experimental import pallas as pl
from jax.experimental.pallas import tpu as pltpu

N = 4096
B = 512
K = 10


def _relu(x):
    return jnp.maximum(x, 0.0)


def _bdot(a, b):
    return jnp.dot(a.astype(jnp.bfloat16), b.astype(jnp.bfloat16),
                   preferred_element_type=jnp.float32)


def _chol_inv_t(gram):
    row = jax.lax.broadcasted_iota(jnp.int32, (K, K), 0)
    col = jax.lax.broadcasted_iota(jnp.int32, (K, K), 1)
    eye = (row == col).astype(jnp.float32)
    A = gram
    L = jnp.zeros((K, K), jnp.float32)
    for k in range(K):
        inv_s = jax.lax.rsqrt(A[k:k + 1, k:k + 1])
        lk = jnp.where(row[:, k:k + 1] >= k,
                       A[:, k:k + 1] * inv_s, 0.0)
        lk_t = jnp.where(col[k:k + 1, :] >= k,
                         A[k:k + 1, :] * inv_s, 0.0)
        L = L + jnp.where(col == k, lk, 0.0)
        A = A - lk * lk_t
    X = jnp.zeros((K, K), jnp.float32)
    for i in range(K):
        acc = jnp.dot(L, X, preferred_element_type=jnp.float32)
        xi = (eye[i:i + 1, :] - acc[i:i + 1, :]) / L[i:i + 1, i:i + 1]
        X = X + jnp.where(row == i, xi, 0.0)
    return X.T


def _fused_kernel(inputs_ref, adj_ref, w0, b0, w1, b1, w2, b2, w3, b3,
                  wg1, wg2, ortho_out, g_out,
                  x1_scr, ab_scr, h_scr, y_scr, gram_scr, ysum_scr, inv_scr):
    p = pl.program_id(0)
    i = pl.program_id(1)

    @pl.when((p == 0) & (i == 0))
    def _():
        x1_scr[...] = jnp.dot(inputs_ref[...], wg1[...],
                              preferred_element_type=jnp.float32
                              ).astype(jnp.bfloat16)

    @pl.when(p == 0)
    def _phase_a():
        x = inputs_ref[pl.ds(i * B, B), :]
        h = _relu(_bdot(x, w0[...]) + b0[...])
        h = _relu(_bdot(h, w1[...]) + b1[...])
        h = _relu(_bdot(h, w2[...]) + b2[...])
        h = jnp.tanh(_bdot(h, w3[...]) + b3[...])
        h_scr[pl.ds(i * B, B), :] = h

        ab = adj_ref[...].astype(jnp.bfloat16)
        ab_scr[pl.ds(i * B, B), :] = ab
        g = _relu(jnp.dot(ab, x1_scr[...],
                          preferred_element_type=jnp.float32))
        y = _bdot(g, wg2[...])
        y_scr[pl.ds(i * B, B), :] = y.astype(jnp.bfloat16)

        gram_blk = jax.lax.dot_general(h, h, (((0,), (0,)), ((), ())),
                                       preferred_element_type=jnp.float32)
        ysum_blk = jnp.sum(y, axis=0, keepdims=True)

        @pl.when(i == 0)
        def _():
            gram_scr[...] = gram_blk
            ysum_scr[...] = ysum_blk

        @pl.when(i > 0)
        def _():
            gram_scr[...] += gram_blk
            ysum_scr[...] += ysum_blk

    @pl.when((p == 1) & (i == 0))
    def _():
        row = jax.lax.broadcasted_iota(jnp.int32, (K, K), 0)
        col = jax.lax.broadcasted_iota(jnp.int32, (K, K), 1)
        gram = gram_scr[...] + 1e-6 * (row == col).astype(jnp.float32)
        inv_scr[...] = _chol_inv_t(gram)

    @pl.when(p == 1)
    def _phase_b():
        hb = h_scr[pl.ds(i * B, B), :]
        ortho_out[...] = 64.0 * jnp.dot(hb, inv_scr[...],
                                        preferred_element_type=jnp.float32)
        g_out[...] = jnp.dot(ab_scr[pl.ds(i * B, B), :], y_scr[...],
                             preferred_element_type=jnp.float32)


@jax.jit
def kernel(inputs, adj, Ws0, bs0, Ws1, bs1, Ws2, bs2, Ws3, bs3, Wg1, Wg2):
    f32 = jnp.float32
    w2p = jnp.pad(Ws2, ((0, 0), (0, 14)))
    b2p = jnp.pad(bs2, (0, 14)).reshape(1, -1)
    w3p = jnp.pad(Ws3, ((0, 14), (0, 0)))
    b0 = bs0.reshape(1, -1)
    b1 = bs1.reshape(1, -1)
    b3 = bs3.reshape(1, -1)

    grid = N // B
    full = lambda s: pl.BlockSpec(s, lambda p, i: (0, 0))
    rows = lambda w: pl.BlockSpec((B, w),
                                  lambda p, i: (jnp.where(p == 0, 0, i), 0))
    adj_spec = pl.BlockSpec(
        (B, N), lambda p, i: (jnp.where(p == 0, i, grid - 1), 0))

    ortho, out_g = pl.pallas_call(
        _fused_kernel,
        grid=(2, grid),
        in_specs=[
            full((N, 128)),
            adj_spec,
            full((128, 1024)), full((1, 1024)),
            full((1024, 512)), full((1, 512)),
            full((512, 64)), full((1, 64)),
            full((64, K)), full((1, K)),
            full((128, 64)),
            full((64, K)),
        ],
        out_specs=[rows(K), rows(K)],
        out_shape=[jax.ShapeDtypeStruct((N, K), f32),
                   jax.ShapeDtypeStruct((N, K), f32)],
        compiler_params=pltpu.CompilerParams(
            vmem_limit_bytes=100 * 1024 * 1024),
        scratch_shapes=[pltpu.VMEM((N, 64), jnp.bfloat16),
                        pltpu.VMEM((N, N), jnp.bfloat16),
                        pltpu.VMEM((N, K), f32),
                        pltpu.VMEM((N, K), jnp.bfloat16),
                        pltpu.VMEM((K, K), f32),
                        pltpu.VMEM((1, K), f32),
                        pltpu.VMEM((K, K), f32)],
    )(inputs, adj, Ws0, b0, Ws1, b1, w2p, b2p, w3p, b3, Wg1, Wg2)

    return (ortho, out_g)


# speedup vs baseline: 1.3739x; 1.0964x over previous
import jax
import jax.numpy as jnp
from jax.experimental import pallas as pl
from jax.experimental.pallas import tpu as pltpu

N = 4096
B = 512
K = 10


def _relu(x):
    return jnp.maximum(x, 0.0)


def _bdot(a, b):
    return jnp.dot(a.astype(jnp.bfloat16), b.astype(jnp.bfloat16),
                   preferred_element_type=jnp.float32)


def _chol_inv_t(gram):
    row = jax.lax.broadcasted_iota(jnp.int32, (K, K), 0)
    col = jax.lax.broadcasted_iota(jnp.int32, (K, K), 1)
    eye = (row == col).astype(jnp.float32)
    A = gram
    L = jnp.zeros((K, K), jnp.float32)
    for k in range(K):
        inv_s = jax.lax.rsqrt(A[k:k + 1, k:k + 1])
        lk = jnp.where(row[:, k:k + 1] >= k,
                       A[:, k:k + 1] * inv_s, 0.0)
        lk_t = jnp.where(col[k:k + 1, :] >= k,
                         A[k:k + 1, :] * inv_s, 0.0)
        L = L + jnp.where(col == k, lk, 0.0)
        A = A - lk * lk_t
    X = jnp.zeros((K, K), jnp.float32)
    for i in range(K):
        acc = jnp.dot(L, X, preferred_element_type=jnp.float32)
        xi = (eye[i:i + 1, :] - acc[i:i + 1, :]) / L[i:i + 1, i:i + 1]
        X = X + jnp.where(row == i, xi, 0.0)
    return X.T


def _fused_kernel(inputs_ref, adj_ref, w0, b0, w1, b1, w2, b2, w3, b3,
                  wg1, wg2, ortho_out, g_out,
                  x1_scr, ab_scr, h_scr, y_scr, gram_scr, inv_scr):
    p = pl.program_id(0)
    i = pl.program_id(1)

    @pl.when((p == 0) & (i == 0))
    def _():
        x1_scr[...] = jnp.dot(inputs_ref[...], wg1[...],
                              preferred_element_type=jnp.float32
                              ).astype(jnp.bfloat16)

    @pl.when((p == 0) & (i % 2 == 0))
    def _mlp_two_blocks():
        x = inputs_ref[pl.ds(i * B, 2 * B), :]
        h = _relu(_bdot(x, w0[...]) + b0[...])
        h = _relu(_bdot(h, w1[...]) + b1[...])
        h = _relu(_bdot(h, w2[...]) + b2[...])
        h = jnp.tanh(_bdot(h, w3[...]) + b3[...])
        h_scr[pl.ds(i * B, 2 * B), :] = h

        gram_blk = jax.lax.dot_general(h, h, (((0,), (0,)), ((), ())),
                                       preferred_element_type=jnp.float32)

        @pl.when(i == 0)
        def _():
            gram_scr[...] = gram_blk

        @pl.when(i > 0)
        def _():
            gram_scr[...] += gram_blk

    @pl.when(p == 0)
    def _phase_a():
        ab = adj_ref[...].astype(jnp.bfloat16)
        ab_scr[pl.ds(i * B, B), :] = ab
        g = _relu(jnp.dot(ab, x1_scr[...],
                          preferred_element_type=jnp.float32))
        y = _bdot(g, wg2[...])
        y_scr[pl.ds(i * B, B), :] = y.astype(jnp.bfloat16)

    @pl.when((p == 1) & (i == 0))
    def _():
        row = jax.lax.broadcasted_iota(jnp.int32, (K, K), 0)
        col = jax.lax.broadcasted_iota(jnp.int32, (K, K), 1)
        gram = gram_scr[...] + 1e-6 * (row == col).astype(jnp.float32)
        inv_scr[...] = _chol_inv_t(gram)

    @pl.when(p == 1)
    def _phase_b():
        hb = h_scr[pl.ds(i * B, B), :]
        ortho_out[...] = 64.0 * jnp.dot(hb, inv_scr[...],
                                        preferred_element_type=jnp.float32)
        g_out[...] = jnp.dot(ab_scr[pl.ds(i * B, B), :], y_scr[...],
                             preferred_element_type=jnp.float32)


@jax.jit
def kernel(inputs, adj, Ws0, bs0, Ws1, bs1, Ws2, bs2, Ws3, bs3, Wg1, Wg2):
    f32 = jnp.float32
    w2p = jnp.pad(Ws2, ((0, 0), (0, 14)))
    b2p = jnp.pad(bs2, (0, 14)).reshape(1, -1)
    w3p = jnp.pad(Ws3, ((0, 14), (0, 0)))
    b0 = bs0.reshape(1, -1)
    b1 = bs1.reshape(1, -1)
    b3 = bs3.reshape(1, -1)

    grid = N // B
    full = lambda s: pl.BlockSpec(s, lambda p, i: (0, 0))
    rows = lambda w: pl.BlockSpec((B, w),
                                  lambda p, i: (jnp.where(p == 0, 0, i), 0))
    adj_spec = pl.BlockSpec(
        (B, N), lambda p, i: (jnp.where(p == 0, i, grid - 1), 0))

    ortho, out_g = pl.pallas_call(
        _fused_kernel,
        grid=(2, grid),
        in_specs=[
            full((N, 128)),
            adj_spec,
            full((128, 1024)), full((1, 1024)),
            full((1024, 512)), full((1, 512)),
            full((512, 64)), full((1, 64)),
            full((64, K)), full((1, K)),
            full((128, 64)),
            full((64, K)),
        ],
        out_specs=[rows(K), rows(K)],
        out_shape=[jax.ShapeDtypeStruct((N, K), f32),
                   jax.ShapeDtypeStruct((N, K), f32)],
        compiler_params=pltpu.CompilerParams(
            vmem_limit_bytes=100 * 1024 * 1024),
        scratch_shapes=[pltpu.VMEM((N, 64), jnp.bfloat16),
                        pltpu.VMEM((N, N), jnp.bfloat16),
                        pltpu.VMEM((N, K), f32),
                        pltpu.VMEM((N, K), jnp.bfloat16),
                        pltpu.VMEM((K, K), f32),
                        pltpu.VMEM((K, K), f32)],
    )(inputs, adj, Ws0, b0, Ws1, b1, w2p, b2p, w3p, b3, Wg1, Wg2)

    return (ortho, out_g)


# speedup vs baseline: 1.3851x; 1.0082x over previous
import jax
import jax.numpy as jnp
from jax.experimental import pallas as pl
from jax.experimental.pallas import tpu as pltpu

N = 4096
B = 512
K = 10


def _relu(x):
    return jnp.maximum(x, 0.0)


def _bdot(a, b):
    return jnp.dot(a.astype(jnp.bfloat16), b.astype(jnp.bfloat16),
                   preferred_element_type=jnp.float32)


def _chol_inv_t(gram):
    row = jax.lax.broadcasted_iota(jnp.int32, (K, K), 0)
    col = jax.lax.broadcasted_iota(jnp.int32, (K, K), 1)
    eye = (row == col).astype(jnp.float32)
    A = gram
    L = jnp.zeros((K, K), jnp.float32)
    for k in range(K):
        inv_s = jax.lax.rsqrt(A[k:k + 1, k:k + 1])
        lk = jnp.where(row[:, k:k + 1] >= k,
                       A[:, k:k + 1] * inv_s, 0.0)
        lk_t = jnp.where(col[k:k + 1, :] >= k,
                         A[k:k + 1, :] * inv_s, 0.0)
        L = L + jnp.where(col == k, lk, 0.0)
        A = A - lk * lk_t
    X = jnp.zeros((K, K), jnp.float32)
    for i in range(K):
        acc = jnp.dot(L, X, preferred_element_type=jnp.float32)
        xi = (eye[i:i + 1, :] - acc[i:i + 1, :]) / L[i:i + 1, i:i + 1]
        X = X + jnp.where(row == i, xi, 0.0)
    return X.T


def _fused_kernel(inputs_ref, adj_ref, w0, b0, w1, b1, w2, b2, w3, b3,
                  wg1, wg2, ortho_out, g_out,
                  x1_scr, ab_scr, h_scr, y_scr, gram_scr, inv_scr):
    p = pl.program_id(0)
    i = pl.program_id(1)

    @pl.when((p == 0) & (i == 0))
    def _():
        x1_scr[...] = jnp.dot(inputs_ref[...], wg1[...],
                              preferred_element_type=jnp.float32
                              ).astype(jnp.bfloat16)

    @pl.when((p == 0) & (i % 2 == 0))
    def _mlp_two_blocks():
        x = inputs_ref[pl.ds(i * B, 2 * B), :]
        h = _relu(_bdot(x, w0[...]) + b0[...])
        h = _relu(_bdot(h, w1[...]) + b1[...])
        h = _relu(_bdot(h, w2[...]) + b2[...])
        h = jnp.tanh(_bdot(h, w3[...]) + b3[...])
        h_scr[pl.ds(i * B, 2 * B), :] = h

        gram_blk = jax.lax.dot_general(h, h, (((0,), (0,)), ((), ())),
                                       preferred_element_type=jnp.float32)

        @pl.when(i == 0)
        def _():
            gram_scr[...] = gram_blk

        @pl.when(i > 0)
        def _():
            gram_scr[...] += gram_blk

    @pl.when(p == 0)
    def _phase_a():
        ab = adj_ref[...].astype(jnp.bfloat16)
        ab_scr[pl.ds(i * B, B), :] = ab
        g = _relu(jnp.dot(ab, x1_scr[...],
                          preferred_element_type=jnp.float32))
        y = _bdot(g, wg2[...])
        y_scr[pl.ds(i * B, B), :] = y.astype(jnp.bfloat16)

    @pl.when((p == 1) & (i == 0))
    def _():
        row = jax.lax.broadcasted_iota(jnp.int32, (K, K), 0)
        col = jax.lax.broadcasted_iota(jnp.int32, (K, K), 1)
        gram = gram_scr[...] + 1e-6 * (row == col).astype(jnp.float32)
        inv_scr[...] = _chol_inv_t(gram)

    @pl.when((p == 1) & (i < 4))
    def _phase_b():
        hb = h_scr[pl.ds(i * 2 * B, 2 * B), :]
        ortho_out[...] = 64.0 * jnp.dot(hb, inv_scr[...],
                                        preferred_element_type=jnp.float32)
        g_out[...] = jnp.dot(ab_scr[pl.ds(i * 2 * B, 2 * B), :], y_scr[...],
                             preferred_element_type=jnp.float32)


@jax.jit
def kernel(inputs, adj, Ws0, bs0, Ws1, bs1, Ws2, bs2, Ws3, bs3, Wg1, Wg2):
    f32 = jnp.float32
    w2p = jnp.pad(Ws2, ((0, 0), (0, 14)))
    b2p = jnp.pad(bs2, (0, 14)).reshape(1, -1)
    w3p = jnp.pad(Ws3, ((0, 14), (0, 0)))
    b0 = bs0.reshape(1, -1)
    b1 = bs1.reshape(1, -1)
    b3 = bs3.reshape(1, -1)

    grid = N // B
    full = lambda s: pl.BlockSpec(s, lambda p, i: (0, 0))
    rows = lambda w: pl.BlockSpec(
        (2 * B, w),
        lambda p, i: (jnp.where(p == 0, 0, jnp.minimum(i, 3)), 0))
    adj_spec = pl.BlockSpec(
        (B, N), lambda p, i: (jnp.where(p == 0, i, grid - 1), 0))

    ortho, out_g = pl.pallas_call(
        _fused_kernel,
        grid=(2, grid),
        in_specs=[
            full((N, 128)),
            adj_spec,
            full((128, 1024)), full((1, 1024)),
            full((1024, 512)), full((1, 512)),
            full((512, 64)), full((1, 64)),
            full((64, K)), full((1, K)),
            full((128, 64)),
            full((64, K)),
        ],
        out_specs=[rows(K), rows(K)],
        out_shape=[jax.ShapeDtypeStruct((N, K), f32),
                   jax.ShapeDtypeStruct((N, K), f32)],
        compiler_params=pltpu.CompilerParams(
            vmem_limit_bytes=100 * 1024 * 1024),
        scratch_shapes=[pltpu.VMEM((N, 64), jnp.bfloat16),
                        pltpu.VMEM((N, N), jnp.bfloat16),
                        pltpu.VMEM((N, K), f32),
                        pltpu.VMEM((N, K), jnp.bfloat16),
                        pltpu.VMEM((K, K), f32),
                        pltpu.VMEM((K, K), f32)],
    )(inputs, adj, Ws0, b0, Ws1, b1, w2p, b2p, w3p, b3, Wg1, Wg2)

    return (ortho, out_g)


# speedup vs baseline: 1.3948x; 1.0070x over previous
import jax
import jax.numpy as jnp
from jax.experimental import pallas as pl
from jax.experimental.pallas import tpu as pltpu

N = 4096
B = 512
K = 10


def _relu(x):
    return jnp.maximum(x, 0.0)


def _bdot(a, b):
    return jnp.dot(a.astype(jnp.bfloat16), b.astype(jnp.bfloat16),
                   preferred_element_type=jnp.float32)


def _chol_inv_t(gram):
    row = jax.lax.broadcasted_iota(jnp.int32, (K, K), 0)
    col = jax.lax.broadcasted_iota(jnp.int32, (K, K), 1)
    eye = (row == col).astype(jnp.float32)
    A = gram
    L = jnp.zeros((K, K), jnp.float32)
    for k in range(K):
        inv_s = jax.lax.rsqrt(A[k:k + 1, k:k + 1])
        lk = jnp.where(row[:, k:k + 1] >= k,
                       A[:, k:k + 1] * inv_s, 0.0)
        lk_t = jnp.where(col[k:k + 1, :] >= k,
                         A[k:k + 1, :] * inv_s, 0.0)
        L = L + jnp.where(col == k, lk, 0.0)
        A = A - lk * lk_t
    X = jnp.zeros((K, K), jnp.float32)
    for i in range(K):
        acc = jnp.dot(L, X, preferred_element_type=jnp.float32)
        xi = (eye[i:i + 1, :] - acc[i:i + 1, :]) / L[i:i + 1, i:i + 1]
        X = X + jnp.where(row == i, xi, 0.0)
    return X.T


def _fused_kernel(inputs_ref, adj_ref, w0, b0, w1, b1, w2, b2, w3, b3,
                  wg1, wg2, ortho_out, g_out,
                  x1_scr, ab_scr, h_scr, y_scr, gram_scr, inv_scr):
    p = pl.program_id(0)
    i = pl.program_id(1)

    @pl.when((p == 0) & (i == 0))
    def _():
        x1_scr[...] = jnp.dot(inputs_ref[...], wg1[...],
                              preferred_element_type=jnp.float32
                              ).astype(jnp.bfloat16)

    @pl.when((p == 0) & (i % 4 == 0))
    def _mlp_two_blocks():
        x = inputs_ref[pl.ds(i * B, 4 * B), :]
        h = _relu(_bdot(x, w0[...]) + b0[...])
        h = _relu(_bdot(h, w1[...]) + b1[...])
        h = _relu(_bdot(h, w2[...]) + b2[...])
        h = jnp.tanh(_bdot(h, w3[...]) + b3[...])
        h_scr[pl.ds(i * B, 4 * B), :] = h.astype(jnp.bfloat16)

        gram_blk = jax.lax.dot_general(h, h, (((0,), (0,)), ((), ())),
                                       preferred_element_type=jnp.float32)

        @pl.when(i == 0)
        def _():
            gram_scr[...] = gram_blk

        @pl.when(i > 0)
        def _():
            gram_scr[...] += gram_blk

    @pl.when(p == 0)
    def _phase_a():
        ab = adj_ref[...].astype(jnp.bfloat16)
        ab_scr[pl.ds(i * B, B), :] = ab
        g = _relu(jnp.dot(ab, x1_scr[...],
                          preferred_element_type=jnp.float32))
        y = _bdot(g, wg2[...])
        y_scr[pl.ds(i * B, B), :] = y.astype(jnp.bfloat16)

    @pl.when((p == 1) & (i == 0))
    def _():
        row = jax.lax.broadcasted_iota(jnp.int32, (K, K), 0)
        col = jax.lax.broadcasted_iota(jnp.int32, (K, K), 1)
        gram = gram_scr[...] + 1e-6 * (row == col).astype(jnp.float32)
        inv_scr[...] = _chol_inv_t(gram)

    @pl.when((p == 1) & (i < 4))
    def _phase_b():
        hb = h_scr[pl.ds(i * 2 * B, 2 * B), :]
        ortho_out[...] = 64.0 * jnp.dot(hb, inv_scr[...],
                                        preferred_element_type=jnp.float32)
        g_out[...] = jnp.dot(ab_scr[pl.ds(i * 2 * B, 2 * B), :], y_scr[...],
                             preferred_element_type=jnp.float32)


@jax.jit
def kernel(inputs, adj, Ws0, bs0, Ws1, bs1, Ws2, bs2, Ws3, bs3, Wg1, Wg2):
    f32 = jnp.float32
    w2p = jnp.pad(Ws2, ((0, 0), (0, 14)))
    b2p = jnp.pad(bs2, (0, 14)).reshape(1, -1)
    w3p = jnp.pad(Ws3, ((0, 14), (0, 0)))
    b0 = bs0.reshape(1, -1)
    b1 = bs1.reshape(1, -1)
    b3 = bs3.reshape(1, -1)

    grid = N // B
    full = lambda s: pl.BlockSpec(s, lambda p, i: (0, 0))
    rows = lambda w: pl.BlockSpec(
        (2 * B, w),
        lambda p, i: (jnp.where(p == 0, 0, jnp.minimum(i, 3)), 0))
    adj_spec = pl.BlockSpec(
        (B, N), lambda p, i: (jnp.where(p == 0, i, grid - 1), 0))

    ortho, out_g = pl.pallas_call(
        _fused_kernel,
        grid=(2, grid),
        in_specs=[
            full((N, 128)),
            adj_spec,
            full((128, 1024)), full((1, 1024)),
            full((1024, 512)), full((1, 512)),
            full((512, 64)), full((1, 64)),
            full((64, K)), full((1, K)),
            full((128, 64)),
            full((64, K)),
        ],
        out_specs=[rows(K), rows(K)],
        out_shape=[jax.ShapeDtypeStruct((N, K), f32),
                   jax.ShapeDtypeStruct((N, K), f32)],
        compiler_params=pltpu.CompilerParams(
            vmem_limit_bytes=100 * 1024 * 1024),
        scratch_shapes=[pltpu.VMEM((N, 64), jnp.bfloat16),
                        pltpu.VMEM((N, N), jnp.bfloat16),
                        pltpu.VMEM((N, K), jnp.bfloat16),
                        pltpu.VMEM((N, K), jnp.bfloat16),
                        pltpu.VMEM((K, K), f32),
                        pltpu.VMEM((K, K), f32)],
    )(inputs, adj, Ws0, b0, Ws1, b1, w2p, b2p, w3p, b3, Wg1, Wg2)

    return (ortho, out_g)
